# Initial kernel scaffold; baseline (speedup 1.0000x reference)
import jax
import jax.numpy as jnp
from jax.experimental import pallas as pl


def kernel(x, H, batch, W, b):
    raise NotImplementedError("write your pallas kernel here")


# speedup vs baseline: 4.2509x; 4.2509x over previous
import functools

import jax
import jax.numpy as jnp
from jax import lax
from jax.experimental import pallas as pl
from jax.experimental.pallas import tpu as pltpu
from jax.experimental.pallas import tpu_sc as plsc

V = 10000
E = 16
D = 128
B = 16

NW = 32
CH = 312
CHBUF = 328
NDB = D // 16


def _sc_segment_sums(x, H, batch):
    mesh = plsc.VectorSubcoreMesh(core_axis_name="c", subcore_axis_name="s")

    @functools.partial(
        pl.kernel,
        out_type=(
            jax.ShapeDtypeStruct((NW, B * E * D), jnp.float32),
            jax.ShapeDtypeStruct((NW, B * E), jnp.float32),
            jax.ShapeDtypeStruct((NW, B), jnp.float32),
        ),
        mesh=mesh,
        scratch_types=[
            pltpu.VMEM((CHBUF, D), jnp.float32),
            pltpu.VMEM((CHBUF, E), jnp.float32),
            pltpu.VMEM((CHBUF + 16,), jnp.int32),
            pltpu.VMEM((B * E * D,), jnp.float32),
            pltpu.VMEM((B * E,), jnp.float32),
            pltpu.VMEM((B,), jnp.float32),
        ],
    )
    def k(x_hbm, h_hbm, b_hbm, mpart_hbm, sh_hbm, cnt_hbm,
          xv, hv, bv, macc, sacc, cacc):
        wid = lax.axis_index("s") * 2 + lax.axis_index("c")
        base = wid * CH
        n = jnp.where(wid == NW - 1, CHBUF, CH)

        pltpu.sync_copy(x_hbm.at[pl.ds(base, CHBUF)], xv)
        pltpu.sync_copy(h_hbm.at[pl.ds(base, CHBUF)], hv)
        pltpu.sync_copy(b_hbm.at[pl.ds(base, CHBUF)], bv.at[pl.ds(0, CHBUF)])

        zero16 = jnp.zeros((16,), jnp.float32)

        def zero_body(j, _):
            macc[pl.ds(j * 16, 16)] = zero16
            return 0

        lax.fori_loop(0, (B * E * D) // 16, zero_body, 0)

        def zero_s(j, _):
            sacc[pl.ds(j * 16, 16)] = zero16
            return 0

        lax.fori_loop(0, (B * E) // 16, zero_s, 0)
        cacc[...] = zero16

        lane = lax.iota(jnp.int32, 16)

        def body(v, _):
            g = bv[pl.ds(v, 16)][0]
            hrow = hv[v, :]
            plsc.addupdate(sacc.at[pl.ds(g * E, E)], hrow)
            onehot = jnp.where(lane == g, 1.0, 0.0).astype(jnp.float32)
            plsc.addupdate(cacc.at[pl.ds(0, B)], onehot)
            gbase = g * (E * D)
            for e in range(E):
                he = hrow[e]
                ebase = gbase + e * D
                for db in range(NDB):
                    xb = xv[v, pl.ds(db * 16, 16)]
                    plsc.addupdate(macc.at[pl.ds(ebase + db * 16, 16)],
                                   he * xb)
            return 0

        lax.fori_loop(0, n, body, 0)

        pltpu.sync_copy(macc, mpart_hbm.at[wid])
        pltpu.sync_copy(sacc, sh_hbm.at[wid])
        pltpu.sync_copy(cacc, cnt_hbm.at[wid])

    return k(x, H, batch)


def _tc_finalize_body(mpart_ref, sh_ref, cnt_ref, w_ref, b_ref,
                      c_ref, he_ref):
    msum = mpart_ref[0]
    for w in range(1, NW):
        msum = msum + mpart_ref[w]
    shf = jnp.sum(sh_ref[...], axis=0, keepdims=True)
    cnt = jnp.sum(cnt_ref[...], axis=0, keepdims=True)
    inv = 1.0 / jnp.maximum(cnt, 1.0)

    jj = lax.broadcasted_iota(jnp.int32, (B * E, B), 0) // E
    gg = lax.broadcasted_iota(jnp.int32, (B * E, B), 1)
    sel = (jj == gg).astype(jnp.float32)
    inv_rows = lax.dot_general(sel, inv, (((1,), (1,)), ((), ())),
                               preferred_element_type=jnp.float32,
                               precision=lax.Precision.HIGHEST)

    e_pre = msum * inv_rows
    edge = lax.dot_general(e_pre, w_ref[...], (((1,), (1,)), ((), ())),
                           preferred_element_type=jnp.float32,
                           precision=lax.Precision.HIGHEST)
    edge = jnp.maximum(edge + b_ref[...], 0.0)
    he_ref[...] = edge

    gg2 = lax.broadcasted_iota(jnp.int32, (B, B * E), 0)
    jj2 = lax.broadcasted_iota(jnp.int32, (B, B * E), 1) // E
    qm = (gg2 == jj2).astype(jnp.float32) * shf
    cu = lax.dot_general(qm, edge, (((1,), (0,)), ((), ())),
                         preferred_element_type=jnp.float32,
                         precision=lax.Precision.HIGHEST)
    ii = lax.broadcasted_iota(jnp.int32, (B, B), 0)
    kk = lax.broadcasted_iota(jnp.int32, (B, B), 1)
    ident = (ii == kk).astype(jnp.float32)
    inv_col = lax.dot_general(ident, inv, (((1,), (1,)), ((), ())),
                              preferred_element_type=jnp.float32,
                              precision=lax.Precision.HIGHEST)
    c_ref[...] = cu * inv_col * (1.0 / E)


def _tc_finalize(mpart, sh, cnt, W, b2):
    return pl.pallas_call(
        _tc_finalize_body,
        out_shape=(
            jax.ShapeDtypeStruct((B, D), jnp.float32),
            jax.ShapeDtypeStruct((B * E, D), jnp.float32),
        ),
    )(mpart, sh, cnt, W, b2)


def kernel(x, H, batch, W, b):
    batch = batch.astype(jnp.int32)
    mpart, sh, cnt = _sc_segment_sums(x, H, batch)
    mpart = mpart.reshape(NW, B * E, D)
    b2 = b.reshape(1, D)
    c, h_e = _tc_finalize(mpart, sh, cnt, W, b2)
    return (c, h_e)


# speedup vs baseline: 13.9095x; 3.2721x over previous
import functools

import jax
import jax.numpy as jnp
from jax import lax
from jax.experimental import pallas as pl
from jax.experimental.pallas import tpu as pltpu
from jax.experimental.pallas import tpu_sc as plsc

V = 10000
E = 16
D = 128
B = 16

NW = 32
CH = 312
CHBUF = 328
NDP = D // 32


def _sc_segment_sums(x, H, batch):
    mesh = plsc.VectorSubcoreMesh(core_axis_name="c", subcore_axis_name="s")

    @functools.partial(
        pl.kernel,
        out_type=(
            jax.ShapeDtypeStruct((NW, B * E * D), jnp.float32),
            jax.ShapeDtypeStruct((NW, B * E), jnp.float32),
            jax.ShapeDtypeStruct((NW, B), jnp.float32),
        ),
        mesh=mesh,
        scratch_types=[
            pltpu.VMEM((CHBUF, D), jnp.float32),
            pltpu.VMEM((CHBUF, E), jnp.float32),
            pltpu.VMEM((CHBUF + 16,), jnp.int32),
            pltpu.VMEM((B * E * D,), jnp.float32),
            pltpu.VMEM((B * E,), jnp.float32),
            pltpu.VMEM((B,), jnp.float32),
        ],
    )
    def k(x_hbm, h_hbm, b_hbm, mpart_hbm, sh_hbm, cnt_hbm,
          xv, hv, bv, macc, sacc, cacc):
        wid = lax.axis_index("s") * 2 + lax.axis_index("c")
        base = wid * CH
        n = jnp.where(wid == NW - 1, CHBUF, CH)

        pltpu.sync_copy(x_hbm.at[pl.ds(base, CHBUF)], xv)
        pltpu.sync_copy(h_hbm.at[pl.ds(base, CHBUF)], hv)
        pltpu.sync_copy(b_hbm.at[pl.ds(base, CHBUF)], bv.at[pl.ds(0, CHBUF)])

        lane = lax.iota(jnp.int32, 16)
        zeros = jnp.zeros((16,), jnp.float32)

        def count_body(v, carry):
            lt, le = carry
            b_v = bv[pl.ds(v, 16)][0]
            lt = lt + jnp.where(b_v < lane, 1, 0)
            le = le + jnp.where(b_v <= lane, 1, 0)
            return (lt, le)

        cnt_lt, cnt_le = lax.fori_loop(
            0, n, count_body,
            (jnp.zeros((16,), jnp.int32), jnp.zeros((16,), jnp.int32)))

        cacc[...] = (cnt_le - cnt_lt).astype(jnp.float32)

        for g in range(B):
            lo = cnt_lt[g]
            hi = cnt_le[g]
            for dp in range(NDP):
                col = dp * 32

                def seg_body(v, accs, _col=col):
                    hrow = hv[v, :]
                    xb0 = xv[v, pl.ds(_col, 16)]
                    xb1 = xv[v, pl.ds(_col + 16, 16)]
                    out = []
                    for e in range(E):
                        he = hrow[e]
                        out.append(accs[2 * e] + he * xb0)
                        out.append(accs[2 * e + 1] + he * xb1)
                    if _col == 0:
                        out.append(accs[2 * E] + hrow)
                    return tuple(out)

                ncar = 2 * E + (1 if dp == 0 else 0)
                accs = lax.fori_loop(lo, hi, seg_body, (zeros,) * ncar)
                gbase = g * (E * D)
                for e in range(E):
                    macc[pl.ds(gbase + e * D + col, 16)] = accs[2 * e]
                    macc[pl.ds(gbase + e * D + col + 16, 16)] = accs[2 * e + 1]
                if dp == 0:
                    sacc[pl.ds(g * E, E)] = accs[2 * E]

        pltpu.sync_copy(macc, mpart_hbm.at[wid])
        pltpu.sync_copy(sacc, sh_hbm.at[wid])
        pltpu.sync_copy(cacc, cnt_hbm.at[wid])

    return k(x, H, batch)


def _tc_finalize_body(mpart_ref, sh_ref, cnt_ref, w_ref, b_ref,
                      c_ref, he_ref):
    msum = mpart_ref[0]
    for w in range(1, NW):
        msum = msum + mpart_ref[w]
    shf = jnp.sum(sh_ref[...], axis=0, keepdims=True)
    cnt = jnp.sum(cnt_ref[...], axis=0, keepdims=True)
    inv = 1.0 / jnp.maximum(cnt, 1.0)

    jj = lax.broadcasted_iota(jnp.int32, (B * E, B), 0) // E
    gg = lax.broadcasted_iota(jnp.int32, (B * E, B), 1)
    sel = (jj == gg).astype(jnp.float32)
    inv_rows = lax.dot_general(sel, inv, (((1,), (1,)), ((), ())),
                               preferred_element_type=jnp.float32,
                               precision=lax.Precision.HIGHEST)

    e_pre = msum * inv_rows
    edge = lax.dot_general(e_pre, w_ref[...], (((1,), (1,)), ((), ())),
                           preferred_element_type=jnp.float32,
                           precision=lax.Precision.HIGHEST)
    edge = jnp.maximum(edge + b_ref[...], 0.0)
    he_ref[...] = edge

    gg2 = lax.broadcasted_iota(jnp.int32, (B, B * E), 0)
    jj2 = lax.broadcasted_iota(jnp.int32, (B, B * E), 1) // E
    qm = (gg2 == jj2).astype(jnp.float32) * shf
    cu = lax.dot_general(qm, edge, (((1,), (0,)), ((), ())),
                         preferred_element_type=jnp.float32,
                         precision=lax.Precision.HIGHEST)
    ii = lax.broadcasted_iota(jnp.int32, (B, B), 0)
    kk = lax.broadcasted_iota(jnp.int32, (B, B), 1)
    ident = (ii == kk).astype(jnp.float32)
    inv_col = lax.dot_general(ident, inv, (((1,), (1,)), ((), ())),
                              preferred_element_type=jnp.float32,
                              precision=lax.Precision.HIGHEST)
    c_ref[...] = cu * inv_col * (1.0 / E)


def _tc_finalize(mpart, sh, cnt, W, b2):
    return pl.pallas_call(
        _tc_finalize_body,
        out_shape=(
            jax.ShapeDtypeStruct((B, D), jnp.float32),
            jax.ShapeDtypeStruct((B * E, D), jnp.float32),
        ),
    )(mpart, sh, cnt, W, b2)


def kernel(x, H, batch, W, b):
    batch = batch.astype(jnp.int32)
    mpart, sh, cnt = _sc_segment_sums(x, H, batch)
    mpart = mpart.reshape(NW, B * E, D)
    b2 = b.reshape(1, D)
    c, h_e = _tc_finalize(mpart, sh, cnt, W, b2)
    return (c, h_e)


# speedup vs baseline: 15.3813x; 1.1058x over previous
import functools

import jax
import jax.numpy as jnp
from jax import lax
from jax.experimental import pallas as pl
from jax.experimental.pallas import tpu as pltpu
from jax.experimental.pallas import tpu_sc as plsc

V = 10000
E = 16
D = 128
B = 16

NW = 32
CH = 312
CHBUF = 328
NDP = D // 32


def _sc_segment_sums(x, H, batch):
    mesh = plsc.VectorSubcoreMesh(core_axis_name="c", subcore_axis_name="s")

    @functools.partial(
        pl.kernel,
        out_type=(
            jax.ShapeDtypeStruct((NW, B * E, D), jnp.float32),
            jax.ShapeDtypeStruct((NW, B * E), jnp.float32),
            jax.ShapeDtypeStruct((NW, B), jnp.float32),
        ),
        mesh=mesh,
        scratch_types=[
            pltpu.VMEM((CHBUF, D), jnp.float32),
            pltpu.VMEM((CHBUF, E), jnp.float32),
            pltpu.VMEM((CHBUF + 16,), jnp.int32),
            pltpu.VMEM((B * E, D), jnp.float32),
            pltpu.VMEM((B * E,), jnp.float32),
            pltpu.VMEM((B,), jnp.float32),
        ],
    )
    def k(x_hbm, h_hbm, b_hbm, mpart_hbm, sh_hbm, cnt_hbm,
          xv, hv, bv, macc, sacc, cacc):
        wid = lax.axis_index("s") * 2 + lax.axis_index("c")
        base = wid * CH
        n = jnp.where(wid == NW - 1, CHBUF, CH)

        pltpu.sync_copy(x_hbm.at[pl.ds(base, CHBUF)], xv)
        pltpu.sync_copy(h_hbm.at[pl.ds(base, CHBUF)], hv)
        pltpu.sync_copy(b_hbm.at[pl.ds(base, CHBUF)], bv.at[pl.ds(0, CHBUF)])

        lane = lax.iota(jnp.int32, 16)
        zeros = jnp.zeros((16,), jnp.float32)

        def count_body(v, carry):
            lt, le = carry
            b_v = bv[pl.ds(v, 16)][0]
            lt = lt + jnp.where(b_v < lane, 1, 0)
            le = le + jnp.where(b_v <= lane, 1, 0)
            return (lt, le)

        cnt_lt, cnt_le = lax.fori_loop(
            0, n, count_body,
            (jnp.zeros((16,), jnp.int32), jnp.zeros((16,), jnp.int32)))

        cacc[...] = (cnt_le - cnt_lt).astype(jnp.float32)

        for g in range(B):
            lo = cnt_lt[g]
            hi = cnt_le[g]
            for e in range(E):
                macc[g * E + e, pl.ds(32, 16)] = zeros
                macc[g * E + e, pl.ds(48, 16)] = zeros
                macc[g * E + e, pl.ds(96, 16)] = zeros
                macc[g * E + e, pl.ds(112, 16)] = zeros
            for dph in range(2):
                col = dph * 64

                def seg_body(v, accs, _col=col, _g=g, _dph=dph):
                    hrow = hv[v, :]
                    xb0 = xv[v, pl.ds(_col, 16)]
                    xb1 = xv[v, pl.ds(_col + 16, 16)]
                    xb2 = xv[v, pl.ds(_col + 32, 16)]
                    xb3 = xv[v, pl.ds(_col + 48, 16)]
                    out = []
                    for e in range(E):
                        he = hrow[e]
                        out.append(accs[2 * e] + he * xb0)
                        out.append(accs[2 * e + 1] + he * xb1)
                        plsc.addupdate(
                            macc.at[_g * E + e, pl.ds(_col + 32, 16)],
                            he * xb2)
                        plsc.addupdate(
                            macc.at[_g * E + e, pl.ds(_col + 48, 16)],
                            he * xb3)
                    if _dph == 0:
                        out.append(accs[2 * E] + hrow)
                    return tuple(out)

                ncar = 2 * E + (1 if dph == 0 else 0)
                accs = lax.fori_loop(lo, hi, seg_body, (zeros,) * ncar)
                for e in range(E):
                    macc[g * E + e, pl.ds(col, 16)] = accs[2 * e]
                    macc[g * E + e, pl.ds(col + 16, 16)] = accs[2 * e + 1]
                if dph == 0:
                    sacc[pl.ds(g * E, E)] = accs[2 * E]

        pltpu.sync_copy(macc, mpart_hbm.at[wid])
        pltpu.sync_copy(sacc, sh_hbm.at[wid])
        pltpu.sync_copy(cacc, cnt_hbm.at[wid])

    return k(x, H, batch)


def _tc_finalize_body(mpart_ref, sh_ref, cnt_ref, w_ref, b_ref,
                      c_ref, he_ref):
    msum = mpart_ref[0]
    for w in range(1, NW):
        msum = msum + mpart_ref[w]
    shf = jnp.sum(sh_ref[...], axis=0, keepdims=True)
    cnt = jnp.sum(cnt_ref[...], axis=0, keepdims=True)
    inv = 1.0 / jnp.maximum(cnt, 1.0)

    jj = lax.broadcasted_iota(jnp.int32, (B * E, B), 0) // E
    gg = lax.broadcasted_iota(jnp.int32, (B * E, B), 1)
    sel = (jj == gg).astype(jnp.float32)
    inv_rows = lax.dot_general(sel, inv, (((1,), (1,)), ((), ())),
                               preferred_element_type=jnp.float32,
                               precision=lax.Precision.HIGHEST)

    e_pre = msum * inv_rows
    edge = lax.dot_general(e_pre, w_ref[...], (((1,), (1,)), ((), ())),
                           preferred_element_type=jnp.float32,
                           precision=lax.Precision.HIGHEST)
    edge = jnp.maximum(edge + b_ref[...], 0.0)
    he_ref[...] = edge

    gg2 = lax.broadcasted_iota(jnp.int32, (B, B * E), 0)
    jj2 = lax.broadcasted_iota(jnp.int32, (B, B * E), 1) // E
    qm = (gg2 == jj2).astype(jnp.float32) * shf
    cu = lax.dot_general(qm, edge, (((1,), (0,)), ((), ())),
                         preferred_element_type=jnp.float32,
                         precision=lax.Precision.HIGHEST)
    ii = lax.broadcasted_iota(jnp.int32, (B, B), 0)
    kk = lax.broadcasted_iota(jnp.int32, (B, B), 1)
    ident = (ii == kk).astype(jnp.float32)
    inv_col = lax.dot_general(ident, inv, (((1,), (1,)), ((), ())),
                              preferred_element_type=jnp.float32,
                              precision=lax.Precision.HIGHEST)
    c_ref[...] = cu * inv_col * (1.0 / E)


def _tc_finalize(mpart, sh, cnt, W, b2):
    return pl.pallas_call(
        _tc_finalize_body,
        out_shape=(
            jax.ShapeDtypeStruct((B, D), jnp.float32),
            jax.ShapeDtypeStruct((B * E, D), jnp.float32),
        ),
    )(mpart, sh, cnt, W, b2)


def kernel(x, H, batch, W, b):
    batch = batch.astype(jnp.int32)
    mpart, sh, cnt = _sc_segment_sums(x, H, batch)
    b2 = b.reshape(1, D)
    c, h_e = _tc_finalize(mpart, sh, cnt, W, b2)
    return (c, h_e)


# speedup vs baseline: 16.2362x; 1.0556x over previous
import functools

import jax
import jax.numpy as jnp
from jax import lax
from jax.experimental import pallas as pl
from jax.experimental.pallas import tpu as pltpu
from jax.experimental.pallas import tpu_sc as plsc

V = 10000
E = 16
D = 128
B = 16

NW = 32
CH = 312
CHBUF = 328
NDP = D // 32


def _sc_segment_sums(x, H, batch):
    mesh = plsc.VectorSubcoreMesh(core_axis_name="c", subcore_axis_name="s")

    @functools.partial(
        pl.kernel,
        out_type=(
            jax.ShapeDtypeStruct((NW, B * E, D), jnp.float32),
            jax.ShapeDtypeStruct((NW, B * E), jnp.float32),
            jax.ShapeDtypeStruct((NW, B), jnp.float32),
        ),
        mesh=mesh,
        scratch_types=[
            pltpu.VMEM((CHBUF, D), jnp.float32),
            pltpu.VMEM((CHBUF, E), jnp.float32),
            pltpu.VMEM((CHBUF + 16,), jnp.int32),
            pltpu.VMEM((B * E, D), jnp.float32),
            pltpu.VMEM((B * E,), jnp.float32),
            pltpu.VMEM((B,), jnp.float32),
            pltpu.SemaphoreType.DMA,
            pltpu.SemaphoreType.DMA,
        ],
    )
    def k(x_hbm, h_hbm, b_hbm, mpart_hbm, sh_hbm, cnt_hbm,
          xv, hv, bv, macc, sacc, cacc, sem_x, sem_h):
        wid = lax.axis_index("s") * 2 + lax.axis_index("c")
        base = wid * CH
        n = jnp.where(wid == NW - 1, CHBUF, CH)

        pltpu.sync_copy(b_hbm.at[pl.ds(base, CHBUF)], bv.at[pl.ds(0, CHBUF)])
        dx = pltpu.async_copy(x_hbm.at[pl.ds(base, CHBUF)], xv, sem_x)
        dh = pltpu.async_copy(h_hbm.at[pl.ds(base, CHBUF)], hv, sem_h)

        lane = lax.iota(jnp.int32, 16)
        zeros = jnp.zeros((16,), jnp.float32)

        def count_body(v, carry):
            lt, le = carry
            b_v = bv[pl.ds(v, 16)][0]
            lt = lt + jnp.where(b_v < lane, 1, 0)
            le = le + jnp.where(b_v <= lane, 1, 0)
            return (lt, le)

        cnt_lt, cnt_le = lax.fori_loop(
            0, n, count_body,
            (jnp.zeros((16,), jnp.int32), jnp.zeros((16,), jnp.int32)))

        cacc[...] = (cnt_le - cnt_lt).astype(jnp.float32)

        for r in range(B * E):
            macc[r, pl.ds(32, 16)] = zeros
            macc[r, pl.ds(48, 16)] = zeros
            macc[r, pl.ds(96, 16)] = zeros
            macc[r, pl.ds(112, 16)] = zeros
        dh.wait()
        dx.wait()

        for g in range(B):
            lo = cnt_lt[g]
            hi = cnt_le[g]
            for dph in range(2):
                col = dph * 64

                def seg_body(v, accs, _col=col, _g=g, _dph=dph):
                    hrow = hv[v, :]
                    xb0 = xv[v, pl.ds(_col, 16)]
                    xb1 = xv[v, pl.ds(_col + 16, 16)]
                    xb2 = xv[v, pl.ds(_col + 32, 16)]
                    xb3 = xv[v, pl.ds(_col + 48, 16)]
                    out = []
                    for e in range(E):
                        he = hrow[e]
                        out.append(accs[2 * e] + he * xb0)
                        out.append(accs[2 * e + 1] + he * xb1)
                        plsc.addupdate(
                            macc.at[_g * E + e, pl.ds(_col + 32, 16)],
                            he * xb2)
                        plsc.addupdate(
                            macc.at[_g * E + e, pl.ds(_col + 48, 16)],
                            he * xb3)
                    if _dph == 0:
                        out.append(accs[2 * E] + hrow)
                    return tuple(out)

                ncar = 2 * E + (1 if dph == 0 else 0)
                accs = lax.fori_loop(lo, hi, seg_body, (zeros,) * ncar)
                for e in range(E):
                    macc[g * E + e, pl.ds(col, 16)] = accs[2 * e]
                    macc[g * E + e, pl.ds(col + 16, 16)] = accs[2 * e + 1]
                if dph == 0:
                    sacc[pl.ds(g * E, E)] = accs[2 * E]

        pltpu.sync_copy(macc, mpart_hbm.at[wid])
        pltpu.sync_copy(sacc, sh_hbm.at[wid])
        pltpu.sync_copy(cacc, cnt_hbm.at[wid])

    return k(x, H, batch)


def _tc_finalize_body(mpart_ref, sh_ref, cnt_ref, w_ref, b_ref,
                      c_ref, he_ref):
    msum = mpart_ref[0]
    for w in range(1, NW):
        msum = msum + mpart_ref[w]
    shf = jnp.sum(sh_ref[...], axis=0, keepdims=True)
    cnt = jnp.sum(cnt_ref[...], axis=0, keepdims=True)
    inv = 1.0 / jnp.maximum(cnt, 1.0)

    jj = lax.broadcasted_iota(jnp.int32, (B * E, B), 0) // E
    gg = lax.broadcasted_iota(jnp.int32, (B * E, B), 1)
    sel = (jj == gg).astype(jnp.float32)
    inv_rows = lax.dot_general(sel, inv, (((1,), (1,)), ((), ())),
                               preferred_element_type=jnp.float32,
                               precision=lax.Precision.HIGHEST)

    e_pre = msum * inv_rows
    edge = lax.dot_general(e_pre, w_ref[...], (((1,), (1,)), ((), ())),
                           preferred_element_type=jnp.float32,
                           precision=lax.Precision.HIGHEST)
    edge = jnp.maximum(edge + b_ref[...], 0.0)
    he_ref[...] = edge

    gg2 = lax.broadcasted_iota(jnp.int32, (B, B * E), 0)
    jj2 = lax.broadcasted_iota(jnp.int32, (B, B * E), 1) // E
    qm = (gg2 == jj2).astype(jnp.float32) * shf
    cu = lax.dot_general(qm, edge, (((1,), (0,)), ((), ())),
                         preferred_element_type=jnp.float32,
                         precision=lax.Precision.HIGHEST)
    ii = lax.broadcasted_iota(jnp.int32, (B, B), 0)
    kk = lax.broadcasted_iota(jnp.int32, (B, B), 1)
    ident = (ii == kk).astype(jnp.float32)
    inv_col = lax.dot_general(ident, inv, (((1,), (1,)), ((), ())),
                              preferred_element_type=jnp.float32,
                              precision=lax.Precision.HIGHEST)
    c_ref[...] = cu * inv_col * (1.0 / E)


def _tc_finalize(mpart, sh, cnt, W, b2):
    return pl.pallas_call(
        _tc_finalize_body,
        out_shape=(
            jax.ShapeDtypeStruct((B, D), jnp.float32),
            jax.ShapeDtypeStruct((B * E, D), jnp.float32),
        ),
    )(mpart, sh, cnt, W, b2)


def kernel(x, H, batch, W, b):
    batch = batch.astype(jnp.int32)
    mpart, sh, cnt = _sc_segment_sums(x, H, batch)
    b2 = b.reshape(1, D)
    c, h_e = _tc_finalize(mpart, sh, cnt, W, b2)
    return (c, h_e)


# speedup vs baseline: 16.4483x; 1.0131x over previous
import functools

import jax
import jax.numpy as jnp
from jax import lax
from jax.experimental import pallas as pl
from jax.experimental.pallas import tpu as pltpu
from jax.experimental.pallas import tpu_sc as plsc

V = 10000
E = 16
D = 128
B = 16

NW = 32
CH = 312
CHBUF = 328
NDP = D // 32


def _sc_segment_sums(x, H, batch):
    mesh = plsc.VectorSubcoreMesh(core_axis_name="c", subcore_axis_name="s")

    @functools.partial(
        pl.kernel,
        out_type=(
            jax.ShapeDtypeStruct((NW, B * E, D), jnp.float32),
            jax.ShapeDtypeStruct((NW, B * E), jnp.float32),
            jax.ShapeDtypeStruct((NW, B), jnp.float32),
        ),
        mesh=mesh,
        scratch_types=[
            pltpu.VMEM((CHBUF, D), jnp.float32),
            pltpu.VMEM((CHBUF, E), jnp.float32),
            pltpu.VMEM((CHBUF + 16,), jnp.int32),
            pltpu.VMEM((B * E, D), jnp.float32),
            pltpu.VMEM((B * E,), jnp.float32),
            pltpu.VMEM((B,), jnp.float32),
            pltpu.SemaphoreType.DMA,
            pltpu.SemaphoreType.DMA,
            pltpu.SemaphoreType.DMA,
        ],
    )
    def k(x_hbm, h_hbm, b_hbm, mpart_hbm, sh_hbm, cnt_hbm,
          xv, hv, bv, macc, sacc, cacc, sem_x, sem_h, sem_out):
        wid = lax.axis_index("s") * 2 + lax.axis_index("c")
        base = wid * CH
        n = jnp.where(wid == NW - 1, CHBUF, CH)

        pltpu.sync_copy(b_hbm.at[pl.ds(base, CHBUF)], bv.at[pl.ds(0, CHBUF)])
        dx = pltpu.async_copy(x_hbm.at[pl.ds(base, CHBUF)], xv, sem_x)
        dh = pltpu.async_copy(h_hbm.at[pl.ds(base, CHBUF)], hv, sem_h)

        lane = lax.iota(jnp.int32, 16)
        zeros = jnp.zeros((16,), jnp.float32)

        def count_body(v, carry):
            lt, le = carry
            b_v = bv[pl.ds(v, 16)][0]
            lt = lt + jnp.where(b_v < lane, 1, 0)
            le = le + jnp.where(b_v <= lane, 1, 0)
            return (lt, le)

        cnt_lt, cnt_le = lax.fori_loop(
            0, n, count_body,
            (jnp.zeros((16,), jnp.int32), jnp.zeros((16,), jnp.int32)))

        cacc[...] = (cnt_le - cnt_lt).astype(jnp.float32)

        for r in range(B * E):
            macc[r, pl.ds(32, 16)] = zeros
            macc[r, pl.ds(48, 16)] = zeros
            macc[r, pl.ds(96, 16)] = zeros
            macc[r, pl.ds(112, 16)] = zeros
        dh.wait()
        dx.wait()

        wb = []
        for g in range(B):
            lo = cnt_lt[g]
            hi = cnt_le[g]
            for dph in range(2):
                col = dph * 64

                def seg_body(v, accs, _col=col, _g=g, _dph=dph):
                    hrow = hv[v, :]
                    xb0 = xv[v, pl.ds(_col, 16)]
                    xb1 = xv[v, pl.ds(_col + 16, 16)]
                    xb2 = xv[v, pl.ds(_col + 32, 16)]
                    xb3 = xv[v, pl.ds(_col + 48, 16)]
                    out = []
                    for e in range(E):
                        he = hrow[e]
                        out.append(accs[2 * e] + he * xb0)
                        out.append(accs[2 * e + 1] + he * xb1)
                        plsc.addupdate(
                            macc.at[_g * E + e, pl.ds(_col + 32, 16)],
                            he * xb2)
                        plsc.addupdate(
                            macc.at[_g * E + e, pl.ds(_col + 48, 16)],
                            he * xb3)
                    if _dph == 0:
                        out.append(accs[2 * E] + hrow)
                    return tuple(out)

                ncar = 2 * E + (1 if dph == 0 else 0)
                accs = lax.fori_loop(lo, hi, seg_body, (zeros,) * ncar)
                for e in range(E):
                    macc[g * E + e, pl.ds(col, 16)] = accs[2 * e]
                    macc[g * E + e, pl.ds(col + 16, 16)] = accs[2 * e + 1]
                if dph == 0:
                    sacc[pl.ds(g * E, E)] = accs[2 * E]
            wb.append(pltpu.async_copy(
                macc.at[pl.ds(g * E, E)],
                mpart_hbm.at[wid, pl.ds(g * E, E)], sem_out))

        pltpu.sync_copy(sacc, sh_hbm.at[wid])
        pltpu.sync_copy(cacc, cnt_hbm.at[wid])
        for d in wb:
            d.wait()

    return k(x, H, batch)


def _tc_finalize_body(mpart_ref, sh_ref, cnt_ref, w_ref, b_ref,
                      c_ref, he_ref):
    msum = mpart_ref[0]
    for w in range(1, NW):
        msum = msum + mpart_ref[w]
    shf = jnp.sum(sh_ref[...], axis=0, keepdims=True)
    cnt = jnp.sum(cnt_ref[...], axis=0, keepdims=True)
    inv = 1.0 / jnp.maximum(cnt, 1.0)

    jj = lax.broadcasted_iota(jnp.int32, (B * E, B), 0) // E
    gg = lax.broadcasted_iota(jnp.int32, (B * E, B), 1)
    sel = (jj == gg).astype(jnp.float32)
    inv_rows = lax.dot_general(sel, inv, (((1,), (1,)), ((), ())),
                               preferred_element_type=jnp.float32,
                               precision=lax.Precision.HIGHEST)

    e_pre = msum * inv_rows
    edge = lax.dot_general(e_pre, w_ref[...], (((1,), (1,)), ((), ())),
                           preferred_element_type=jnp.float32,
                           precision=lax.Precision.HIGHEST)
    edge = jnp.maximum(edge + b_ref[...], 0.0)
    he_ref[...] = edge

    gg2 = lax.broadcasted_iota(jnp.int32, (B, B * E), 0)
    jj2 = lax.broadcasted_iota(jnp.int32, (B, B * E), 1) // E
    qm = (gg2 == jj2).astype(jnp.float32) * shf
    cu = lax.dot_general(qm, edge, (((1,), (0,)), ((), ())),
                         preferred_element_type=jnp.float32,
                         precision=lax.Precision.HIGHEST)
    ii = lax.broadcasted_iota(jnp.int32, (B, B), 0)
    kk = lax.broadcasted_iota(jnp.int32, (B, B), 1)
    ident = (ii == kk).astype(jnp.float32)
    inv_col = lax.dot_general(ident, inv, (((1,), (1,)), ((), ())),
                              preferred_element_type=jnp.float32,
                              precision=lax.Precision.HIGHEST)
    c_ref[...] = cu * inv_col * (1.0 / E)


def _tc_finalize(mpart, sh, cnt, W, b2):
    return pl.pallas_call(
        _tc_finalize_body,
        out_shape=(
            jax.ShapeDtypeStruct((B, D), jnp.float32),
            jax.ShapeDtypeStruct((B * E, D), jnp.float32),
        ),
    )(mpart, sh, cnt, W, b2)


def kernel(x, H, batch, W, b):
    batch = batch.astype(jnp.int32)
    mpart, sh, cnt = _sc_segment_sums(x, H, batch)
    b2 = b.reshape(1, D)
    c, h_e = _tc_finalize(mpart, sh, cnt, W, b2)
    return (c, h_e)


# speedup vs baseline: 20.7033x; 1.2587x over previous
import functools

import jax
import jax.numpy as jnp
from jax import lax
from jax.experimental import pallas as pl
from jax.experimental.pallas import tpu as pltpu
from jax.experimental.pallas import tpu_sc as plsc

V = 10000
E = 16
D = 128
B = 16

NW = 32
CH = 312
CHBUF = 328
NDP = D // 32


def _sc_segment_sums(x, H, batch):
    mesh = plsc.VectorSubcoreMesh(core_axis_name="c", subcore_axis_name="s")

    @functools.partial(
        pl.kernel,
        out_type=(
            jax.ShapeDtypeStruct((NW, B * E, D), jnp.float32),
            jax.ShapeDtypeStruct((NW, B * E), jnp.float32),
            jax.ShapeDtypeStruct((NW, B), jnp.float32),
        ),
        mesh=mesh,
        scratch_types=[
            pltpu.VMEM((CHBUF, D), jnp.float32),
            pltpu.VMEM((CHBUF, E), jnp.float32),
            pltpu.VMEM((CHBUF + 16,), jnp.int32),
            pltpu.VMEM((B * E, D), jnp.float32),
            pltpu.VMEM((B * E,), jnp.float32),
            pltpu.VMEM((B,), jnp.float32),
            pltpu.SemaphoreType.DMA,
            pltpu.SemaphoreType.DMA,
            pltpu.SemaphoreType.DMA,
            pltpu.SMEM((B,), jnp.int32),
            pltpu.SMEM((B,), jnp.int32),
        ],
    )
    def k(x_hbm, h_hbm, b_hbm, mpart_hbm, sh_hbm, cnt_hbm,
          xv, hv, bv, macc, sacc, cacc, sem_x, sem_h, sem_out,
          slo, shi):
        wid = lax.axis_index("s") * 2 + lax.axis_index("c")
        base = wid * CH
        n = jnp.where(wid == NW - 1, CHBUF, CH)

        pltpu.sync_copy(b_hbm.at[pl.ds(base, CHBUF)], bv.at[pl.ds(0, CHBUF)])
        dx = pltpu.async_copy(x_hbm.at[pl.ds(base, CHBUF)], xv, sem_x)
        dh = pltpu.async_copy(h_hbm.at[pl.ds(base, CHBUF)], hv, sem_h)

        lane = lax.iota(jnp.int32, 16)
        zeros = jnp.zeros((16,), jnp.float32)

        def count_body(v, carry):
            lt, le = carry
            b_v = bv[pl.ds(v, 16)][0]
            lt = lt + jnp.where(b_v < lane, 1, 0)
            le = le + jnp.where(b_v <= lane, 1, 0)
            return (lt, le)

        cnt_lt, cnt_le = lax.fori_loop(
            0, n, count_body,
            (jnp.zeros((16,), jnp.int32), jnp.zeros((16,), jnp.int32)))

        cacc[...] = (cnt_le - cnt_lt).astype(jnp.float32)

        for r in range(B * E):
            macc[r, pl.ds(32, 16)] = zeros
            macc[r, pl.ds(48, 16)] = zeros
            macc[r, pl.ds(96, 16)] = zeros
            macc[r, pl.ds(112, 16)] = zeros
        dh.wait()
        dx.wait()

        for i in range(B):
            slo[i] = cnt_lt[i]
            shi[i] = cnt_le[i]

        def g_body(g, _):
            gE = g * E
            lo = slo[g]
            hi = shi[g]
            for dph in range(2):
                col = dph * 64

                def seg_body(v, accs, _col=col, _gE=gE, _dph=dph):
                    hrow = hv[v, :]
                    xb0 = xv[v, pl.ds(_col, 16)]
                    xb1 = xv[v, pl.ds(_col + 16, 16)]
                    xb2 = xv[v, pl.ds(_col + 32, 16)]
                    xb3 = xv[v, pl.ds(_col + 48, 16)]
                    out = []
                    for e in range(E):
                        he = hrow[e]
                        out.append(accs[2 * e] + he * xb0)
                        out.append(accs[2 * e + 1] + he * xb1)
                        plsc.addupdate(
                            macc.at[_gE + e, pl.ds(_col + 32, 16)],
                            he * xb2)
                        plsc.addupdate(
                            macc.at[_gE + e, pl.ds(_col + 48, 16)],
                            he * xb3)
                    if _dph == 0:
                        out.append(accs[2 * E] + hrow)
                    return tuple(out)

                ncar = 2 * E + (1 if dph == 0 else 0)
                accs = lax.fori_loop(lo, hi, seg_body, (zeros,) * ncar)
                for e in range(E):
                    macc[gE + e, pl.ds(col, 16)] = accs[2 * e]
                    macc[gE + e, pl.ds(col + 16, 16)] = accs[2 * e + 1]
                if dph == 0:
                    sacc[pl.ds(gE, E)] = accs[2 * E]
            pltpu.async_copy(macc.at[pl.ds(gE, E)],
                             mpart_hbm.at[wid, pl.ds(gE, E)], sem_out)
            return 0

        lax.fori_loop(0, B, g_body, 0)

        pltpu.sync_copy(sacc, sh_hbm.at[wid])
        pltpu.sync_copy(cacc, cnt_hbm.at[wid])
        pltpu.make_async_copy(mpart_hbm.at[wid], macc, sem_out).wait()

    return k(x, H, batch)


def _tc_finalize_body(mpart_ref, sh_ref, cnt_ref, w_ref, b_ref,
                      c_ref, he_ref):
    msum = mpart_ref[0]
    for w in range(1, NW):
        msum = msum + mpart_ref[w]
    shf = jnp.sum(sh_ref[...], axis=0, keepdims=True)
    cnt = jnp.sum(cnt_ref[...], axis=0, keepdims=True)
    inv = 1.0 / jnp.maximum(cnt, 1.0)

    jj = lax.broadcasted_iota(jnp.int32, (B * E, B), 0) // E
    gg = lax.broadcasted_iota(jnp.int32, (B * E, B), 1)
    sel = (jj == gg).astype(jnp.float32)
    inv_rows = lax.dot_general(sel, inv, (((1,), (1,)), ((), ())),
                               preferred_element_type=jnp.float32,
                               precision=lax.Precision.HIGHEST)

    e_pre = msum * inv_rows
    edge = lax.dot_general(e_pre, w_ref[...], (((1,), (1,)), ((), ())),
                           preferred_element_type=jnp.float32,
                           precision=lax.Precision.HIGHEST)
    edge = jnp.maximum(edge + b_ref[...], 0.0)
    he_ref[...] = edge

    gg2 = lax.broadcasted_iota(jnp.int32, (B, B * E), 0)
    jj2 = lax.broadcasted_iota(jnp.int32, (B, B * E), 1) // E
    qm = (gg2 == jj2).astype(jnp.float32) * shf
    cu = lax.dot_general(qm, edge, (((1,), (0,)), ((), ())),
                         preferred_element_type=jnp.float32,
                         precision=lax.Precision.HIGHEST)
    ii = lax.broadcasted_iota(jnp.int32, (B, B), 0)
    kk = lax.broadcasted_iota(jnp.int32, (B, B), 1)
    ident = (ii == kk).astype(jnp.float32)
    inv_col = lax.dot_general(ident, inv, (((1,), (1,)), ((), ())),
                              preferred_element_type=jnp.float32,
                              precision=lax.Precision.HIGHEST)
    c_ref[...] = cu * inv_col * (1.0 / E)


def _tc_finalize(mpart, sh, cnt, W, b2):
    return pl.pallas_call(
        _tc_finalize_body,
        out_shape=(
            jax.ShapeDtypeStruct((B, D), jnp.float32),
            jax.ShapeDtypeStruct((B * E, D), jnp.float32),
        ),
    )(mpart, sh, cnt, W, b2)


def kernel(x, H, batch, W, b):
    batch = batch.astype(jnp.int32)
    mpart, sh, cnt = _sc_segment_sums(x, H, batch)
    b2 = b.reshape(1, D)
    c, h_e = _tc_finalize(mpart, sh, cnt, W, b2)
    return (c, h_e)


# speedup vs baseline: 21.5422x; 1.0405x over previous
import functools

import jax
import jax.numpy as jnp
from jax import lax
from jax.experimental import pallas as pl
from jax.experimental.pallas import tpu as pltpu
from jax.experimental.pallas import tpu_sc as plsc

V = 10000
E = 16
D = 128
B = 16

NW = 32
CH = 312
CHBUF = 328
NDP = D // 32


def _sc_segment_sums(x, H, batch):
    mesh = plsc.VectorSubcoreMesh(core_axis_name="c", subcore_axis_name="s")

    @functools.partial(
        pl.kernel,
        out_type=(
            jax.ShapeDtypeStruct((NW, B * E, D), jnp.float32),
            jax.ShapeDtypeStruct((NW, B * E), jnp.float32),
            jax.ShapeDtypeStruct((NW, B), jnp.float32),
        ),
        mesh=mesh,
        scratch_types=[
            pltpu.VMEM((CHBUF, D), jnp.float32),
            pltpu.VMEM((CHBUF, E), jnp.float32),
            pltpu.VMEM((CHBUF + 16,), jnp.int32),
            pltpu.VMEM((B * E, D), jnp.float32),
            pltpu.VMEM((B * E,), jnp.float32),
            pltpu.VMEM((B,), jnp.float32),
            pltpu.SemaphoreType.DMA,
            pltpu.SemaphoreType.DMA,
            pltpu.SemaphoreType.DMA,
            pltpu.SMEM((B,), jnp.int32),
            pltpu.SMEM((B,), jnp.int32),
        ],
    )
    def k(x_hbm, h_hbm, b_hbm, mpart_hbm, sh_hbm, cnt_hbm,
          xv, hv, bv, macc, sacc, cacc, sem_x, sem_h, sem_out,
          slo, shi):
        wid = lax.axis_index("s") * 2 + lax.axis_index("c")
        base = wid * CH
        n = jnp.where(wid == NW - 1, CHBUF, CH)

        pltpu.sync_copy(b_hbm.at[pl.ds(base, CHBUF)], bv.at[pl.ds(0, CHBUF)])
        dx = pltpu.async_copy(x_hbm.at[pl.ds(base, CHBUF)], xv, sem_x)
        dh = pltpu.async_copy(h_hbm.at[pl.ds(base, CHBUF)], hv, sem_h)

        lane = lax.iota(jnp.int32, 16)
        zeros = jnp.zeros((16,), jnp.float32)

        def count_body(v, carry):
            lt, le = carry
            b_v = bv[pl.ds(v, 16)][0]
            lt = lt + jnp.where(b_v < lane, 1, 0)
            le = le + jnp.where(b_v <= lane, 1, 0)
            return (lt, le)

        cnt_lt, cnt_le = lax.fori_loop(
            0, n, count_body,
            (jnp.zeros((16,), jnp.int32), jnp.zeros((16,), jnp.int32)))

        cacc[...] = (cnt_le - cnt_lt).astype(jnp.float32)

        def zero_body(r, _):
            macc[r, pl.ds(32, 16)] = zeros
            macc[r, pl.ds(48, 16)] = zeros
            macc[r, pl.ds(96, 16)] = zeros
            macc[r, pl.ds(112, 16)] = zeros
            return 0

        lax.fori_loop(0, B * E, zero_body, 0)
        dh.wait()
        dx.wait()

        for i in range(B):
            slo[i] = cnt_lt[i]
            shi[i] = cnt_le[i]

        def g_body(g, _):
            gE = g * E
            lo = slo[g]
            hi = shi[g]
            for dph in range(2):
                col = dph * 64

                def seg_body(v, accs, _col=col, _gE=gE, _dph=dph):
                    hrow = hv[v, :]
                    xb0 = xv[v, pl.ds(_col, 16)]
                    xb1 = xv[v, pl.ds(_col + 16, 16)]
                    xb2 = xv[v, pl.ds(_col + 32, 16)]
                    xb3 = xv[v, pl.ds(_col + 48, 16)]
                    out = []
                    for e in range(E):
                        he = hrow[e]
                        out.append(accs[2 * e] + he * xb0)
                        out.append(accs[2 * e + 1] + he * xb1)
                        plsc.addupdate(
                            macc.at[_gE + e, pl.ds(_col + 32, 16)],
                            he * xb2)
                        plsc.addupdate(
                            macc.at[_gE + e, pl.ds(_col + 48, 16)],
                            he * xb3)
                    if _dph == 0:
                        out.append(accs[2 * E] + hrow)
                    return tuple(out)

                ncar = 2 * E + (1 if dph == 0 else 0)
                accs = lax.fori_loop(lo, hi, seg_body, (zeros,) * ncar)
                for e in range(E):
                    macc[gE + e, pl.ds(col, 16)] = accs[2 * e]
                    macc[gE + e, pl.ds(col + 16, 16)] = accs[2 * e + 1]
                if dph == 0:
                    sacc[pl.ds(gE, E)] = accs[2 * E]
            pltpu.async_copy(macc.at[pl.ds(gE, E)],
                             mpart_hbm.at[wid, pl.ds(gE, E)], sem_out)
            return 0

        lax.fori_loop(0, B, g_body, 0)

        pltpu.sync_copy(sacc, sh_hbm.at[wid])
        pltpu.sync_copy(cacc, cnt_hbm.at[wid])
        pltpu.make_async_copy(mpart_hbm.at[wid], macc, sem_out).wait()

    return k(x, H, batch)


def _tc_finalize_body(mpart_ref, sh_ref, cnt_ref, w_ref, b_ref,
                      c_ref, he_ref):
    msum = mpart_ref[0]
    for w in range(1, NW):
        msum = msum + mpart_ref[w]
    shf = jnp.sum(sh_ref[...], axis=0, keepdims=True)
    cnt = jnp.sum(cnt_ref[...], axis=0, keepdims=True)
    inv = 1.0 / jnp.maximum(cnt, 1.0)

    jj = lax.broadcasted_iota(jnp.int32, (B * E, B), 0) // E
    gg = lax.broadcasted_iota(jnp.int32, (B * E, B), 1)
    sel = (jj == gg).astype(jnp.float32)
    inv_rows = lax.dot_general(sel, inv, (((1,), (1,)), ((), ())),
                               preferred_element_type=jnp.float32,
                               precision=lax.Precision.HIGHEST)

    e_pre = msum * inv_rows
    edge = lax.dot_general(e_pre, w_ref[...], (((1,), (1,)), ((), ())),
                           preferred_element_type=jnp.float32,
                           precision=lax.Precision.HIGHEST)
    edge = jnp.maximum(edge + b_ref[...], 0.0)
    he_ref[...] = edge

    gg2 = lax.broadcasted_iota(jnp.int32, (B, B * E), 0)
    jj2 = lax.broadcasted_iota(jnp.int32, (B, B * E), 1) // E
    qm = (gg2 == jj2).astype(jnp.float32) * shf
    cu = lax.dot_general(qm, edge, (((1,), (0,)), ((), ())),
                         preferred_element_type=jnp.float32,
                         precision=lax.Precision.HIGHEST)
    ii = lax.broadcasted_iota(jnp.int32, (B, B), 0)
    kk = lax.broadcasted_iota(jnp.int32, (B, B), 1)
    ident = (ii == kk).astype(jnp.float32)
    inv_col = lax.dot_general(ident, inv, (((1,), (1,)), ((), ())),
                              preferred_element_type=jnp.float32,
                              precision=lax.Precision.HIGHEST)
    c_ref[...] = cu * inv_col * (1.0 / E)


def _tc_finalize(mpart, sh, cnt, W, b2):
    return pl.pallas_call(
        _tc_finalize_body,
        out_shape=(
            jax.ShapeDtypeStruct((B, D), jnp.float32),
            jax.ShapeDtypeStruct((B * E, D), jnp.float32),
        ),
    )(mpart, sh, cnt, W, b2)


def kernel(x, H, batch, W, b):
    batch = batch.astype(jnp.int32)
    mpart, sh, cnt = _sc_segment_sums(x, H, batch)
    b2 = b.reshape(1, D)
    c, h_e = _tc_finalize(mpart, sh, cnt, W, b2)
    return (c, h_e)
